# Initial kernel scaffold; baseline (speedup 1.0000x reference)
import jax
import jax.numpy as jnp
from jax.experimental import pallas as pl


def kernel(user, item, edge_index, edge_weight, user_emb, item_emb):
    raise NotImplementedError("write your pallas kernel here")


# speedup vs baseline: 2.4104x; 2.4104x over previous
import functools

import jax
import jax.numpy as jnp
from jax import lax
from jax.experimental import pallas as pl
from jax.experimental.pallas import tpu as pltpu
from jax.experimental.pallas import tpu_sc as plsc

USER_NUM = 25000
ITEM_NUM = 25000
N_NODES = USER_NUM + ITEM_NUM
N_EDGES = 800000
D = 64
ALPHA = (0.3334, 0.3333, 0.3333)
BATCH = 4096

NSC = 2
NTILE = 16
HALF = 25000
HALF_PAD = 25088
NPAD = 2 * HALF_PAD
ROWS_PER_TILE = HALF_PAD // NTILE
CHUNK = 128
EDGES_PER_TILE = 50048
NCHUNK = EDGES_PER_TILE // CHUNK
EDGES_PAD = NTILE * EDGES_PER_TILE

_mesh = plsc.VectorSubcoreMesh(core_axis_name="c", subcore_axis_name="s")


@functools.partial(
    pl.kernel,
    out_type=jax.ShapeDtypeStruct((NPAD, D), jnp.float32),
    mesh=_mesh,
    compiler_params=pltpu.CompilerParams(needs_layout_passes=False,
                                         use_tc_tiling_on_sc=False),
    scratch_types=[
        pltpu.VMEM((1, CHUNK), jnp.int32),
        pltpu.VMEM((CHUNK,), jnp.int32),
        pltpu.VMEM((CHUNK,), jnp.float32),
        pltpu.VMEM((CHUNK, D), jnp.float32),
        pltpu.VMEM_SHARED((HALF_PAD, D), jnp.float32),
        pltpu.SemaphoreType.DMA,
    ],
)
def _layer_step(emb, dstr, srcr, wr, zrows, out, dstb, srcb, wb, rowsb, acc,
                sem):
    c = lax.axis_index("c")
    s = lax.axis_index("s")
    base = c * HALF
    r0 = s * ROWS_PER_TILE

    pltpu.sync_copy(zrows, acc.at[pl.ds(r0, ROWS_PER_TILE)])
    plsc.subcore_barrier()

    @pl.loop(0, NCHUNK)
    def _block(b):
        pltpu.sync_copy(dstr.at[s, b], dstb.at[0])
        pltpu.sync_copy(srcr.at[s, b], srcb)
        pltpu.sync_copy(wr.at[s, b], wb)

        for j in range(CHUNK // 16):
            sl = pl.ds(j * 16, 16)
            d = dstb[0, sl] - base
            ok = (d >= 0) & (d < HALF)
            dstb[0, sl] = jnp.where(ok, d, HALF)
            sv = srcb[sl]
            srcb[sl] = jnp.where(sv >= HALF, sv + (HALF_PAD - HALF), sv)

        pltpu.async_copy(emb.at[srcb], rowsb, sem).wait()

        @pl.loop(0, CHUNK)
        def _scale(e):
            wv = plsc.load_gather(wb, [jnp.full((16,), e, jnp.int32)])
            for j in range(D // 16):
                sl = pl.ds(j * 16, 16)
                rowsb[e, sl] = rowsb[e, sl] * wv

        pltpu.sync_copy(rowsb, acc.at[dstb.at[0]], add=True)

    plsc.subcore_barrier()
    pltpu.sync_copy(acc.at[pl.ds(r0, ROWS_PER_TILE)],
                    out.at[pl.ds(c * HALF_PAD + r0, ROWS_PER_TILE)])


@functools.partial(
    pl.kernel,
    out_type=jax.ShapeDtypeStruct((BATCH,), jnp.float32),
    mesh=_mesh,
    compiler_params=pltpu.CompilerParams(needs_layout_passes=False,
                                         use_tc_tiling_on_sc=False),
    scratch_types=[
        pltpu.VMEM((BATCH // (NSC * NTILE),), jnp.int32),
        pltpu.VMEM((BATCH // (NSC * NTILE),), jnp.int32),
        pltpu.VMEM((BATCH // (NSC * NTILE), D), jnp.float32),
        pltpu.VMEM((BATCH // (NSC * NTILE), D), jnp.float32),
        pltpu.VMEM((BATCH // (NSC * NTILE), D), jnp.float32),
        pltpu.VMEM((BATCH // (NSC * NTILE), D), jnp.float32),
        pltpu.VMEM((BATCH // (NSC * NTILE), D), jnp.float32),
        pltpu.VMEM((BATCH // (NSC * NTILE), D), jnp.float32),
        pltpu.VMEM((BATCH // (NSC * NTILE),), jnp.float32),
        pltpu.SemaphoreType.DMA,
    ],
)
def _final_step(e0, e1, e2, uix_hbm, iix_hbm, out, uix, iix, u0, u1, u2,
                i0, i1, i2, outb, sem):
    c = lax.axis_index("c")
    s = lax.axis_index("s")
    wid = s * NSC + c
    per = BATCH // (NSC * NTILE)
    qbase = wid * per

    pltpu.sync_copy(uix_hbm.at[pl.ds(qbase, per)], uix)
    pltpu.sync_copy(iix_hbm.at[pl.ds(qbase, per)], iix)

    cps = [
        pltpu.async_copy(e0.at[uix], u0, sem),
        pltpu.async_copy(e1.at[uix], u1, sem),
        pltpu.async_copy(e2.at[uix], u2, sem),
        pltpu.async_copy(e0.at[iix], i0, sem),
        pltpu.async_copy(e1.at[iix], i1, sem),
        pltpu.async_copy(e2.at[iix], i2, sem),
    ]
    for cp in cps:
        cp.wait()

    lanes = lax.iota(jnp.int32, 16)

    @pl.loop(0, per // 16)
    def _group(g):
        gvec = jnp.zeros((16,), jnp.float32)
        for q in range(16):
            e = g * 16 + q
            dv = jnp.zeros((16,), jnp.float32)
            for j in range(D // 16):
                sl = pl.ds(j * 16, 16)
                su = (ALPHA[0] * u0[e, sl] + ALPHA[1] * u1[e, sl]
                      + ALPHA[2] * u2[e, sl])
                si = (ALPHA[0] * i0[e, sl] + ALPHA[1] * i1[e, sl]
                      + ALPHA[2] * i2[e, sl])
                dv = dv + su * si
            gamma = jnp.sum(dv)
            gvec = jnp.where(lanes == q, gamma, gvec)
        sig = 1.0 / (1.0 + jnp.exp(-gvec))
        outb[pl.ds(g * 16, 16)] = sig

    pltpu.sync_copy(outb, out.at[pl.ds(qbase, per)])


@jax.jit
def kernel(user, item, edge_index, edge_weight, user_emb, item_emb):
    zpad = jnp.zeros((HALF_PAD - HALF, D), jnp.float32)
    e0 = jnp.concatenate([user_emb, zpad, item_emb, zpad], axis=0)

    dst = edge_index[0].astype(jnp.int32)
    src = edge_index[1].astype(jnp.int32)
    w = edge_weight.astype(jnp.float32)
    pad = EDGES_PAD - N_EDGES
    dst = jnp.concatenate([dst, jnp.full((pad,), N_NODES + 8192, jnp.int32)])
    src = jnp.concatenate([src, jnp.zeros((pad,), jnp.int32)])
    w = jnp.concatenate([w, jnp.zeros((pad,), jnp.float32)])
    dstr = dst.reshape(NTILE, NCHUNK, CHUNK)
    srcr = src.reshape(NTILE, NCHUNK, CHUNK)
    wr = w.reshape(NTILE, NCHUNK, CHUNK)
    zrows = jnp.zeros((ROWS_PER_TILE, D), jnp.float32)

    e1 = _layer_step(e0, dstr, srcr, wr, zrows)
    e2 = _layer_step(e1, dstr, srcr, wr, zrows)

    uix = user.astype(jnp.int32)
    iix = item.astype(jnp.int32) + HALF_PAD
    return _final_step(e0, e1, e2, uix, iix)


# speedup vs baseline: 3.6186x; 1.5013x over previous
import functools

import jax
import jax.numpy as jnp
from jax import lax
from jax.experimental import pallas as pl
from jax.experimental.pallas import tpu as pltpu
from jax.experimental.pallas import tpu_sc as plsc

USER_NUM = 25000
ITEM_NUM = 25000
N_NODES = USER_NUM + ITEM_NUM
N_EDGES = 800000
D = 64
ALPHA = (0.3334, 0.3333, 0.3333)
BATCH = 4096

NSC = 2
NTILE = 16
HALF = 25000
HALF_PAD = 25088
NPAD = 2 * HALF_PAD
ROWS_PER_TILE = HALF_PAD // NTILE
CHUNK = 128
GRP = 6
NGRP = 66
EDGES_PER_TILE = NGRP * GRP * CHUNK
EDGES_PAD = NTILE * EDGES_PER_TILE
NB = 3

_mesh = plsc.VectorSubcoreMesh(core_axis_name="c", subcore_axis_name="s")


@functools.partial(
    pl.kernel,
    out_type=jax.ShapeDtypeStruct((NPAD, D), jnp.float32),
    mesh=_mesh,
    compiler_params=pltpu.CompilerParams(needs_layout_passes=False,
                                         use_tc_tiling_on_sc=False),
    scratch_types=[
        pltpu.VMEM((2, GRP, CHUNK), jnp.int32),
        pltpu.VMEM((2, GRP, CHUNK), jnp.int32),
        pltpu.VMEM((2, GRP, CHUNK), jnp.float32),
        pltpu.VMEM((NB, CHUNK, D), jnp.float32),
        pltpu.VMEM_SHARED((HALF_PAD, D), jnp.float32),
        pltpu.SemaphoreType.DMA,
        pltpu.SemaphoreType.DMA((NB,)),
        pltpu.SemaphoreType.DMA((NB,)),
    ],
)
def _layer_step(emb, dstr, srcr, wr, zrows, out, dstb, srcb, wb, rowsb, acc,
                isem, gsem, ssem):
    c = lax.axis_index("c")
    s = lax.axis_index("s")
    base = c * HALF
    r0 = s * ROWS_PER_TILE

    def idx_start(g, slot):
        pltpu.async_copy(dstr.at[s, g], dstb.at[slot], isem)
        pltpu.async_copy(srcr.at[s, g], srcb.at[slot], isem)
        pltpu.async_copy(wr.at[s, g], wb.at[slot], isem)

    def idx_wait(slot):
        pltpu.make_async_copy(dstr.at[s, 0], dstb.at[slot], isem).wait()
        pltpu.make_async_copy(srcr.at[s, 0], srcb.at[slot], isem).wait()
        pltpu.make_async_copy(wr.at[s, 0], wb.at[slot], isem).wait()

    def gather_start(slot, srow):
        pltpu.async_copy(emb.at[srow], rowsb.at[slot], gsem.at[slot])

    def gather_wait(slot, srow):
        pltpu.make_async_copy(emb.at[srow], rowsb.at[slot],
                              gsem.at[slot]).wait()

    def scat_start(slot, drow):
        pltpu.async_copy(rowsb.at[slot], acc.at[drow], ssem.at[slot],
                         add=True)

    def scat_wait(slot, drow):
        pltpu.make_async_copy(rowsb.at[slot], acc.at[drow],
                              ssem.at[slot]).wait()

    pltpu.sync_copy(zrows, acc.at[pl.ds(r0, ROWS_PER_TILE)])
    plsc.subcore_barrier()

    idx_start(0, 0)

    def group(g, slot):
        idx_wait(slot)
        nxt = pl.when(g + 1 < NGRP)(lambda: idx_start(g + 1, slot ^ 1))
        del nxt

        for j in range(GRP):
            for k in range(CHUNK // 16):
                sl = pl.ds(k * 16, 16)
                d = dstb[slot, j, sl] - base
                ok = (d >= 0) & (d < HALF)
                dstb[slot, j, sl] = jnp.where(ok, d, HALF)
                sv = srcb[slot, j, sl]
                srcb[slot, j, sl] = jnp.where(sv >= HALF,
                                              sv + (HALF_PAD - HALF), sv)
            if j < NB:
                gather_start(j, srcb.at[slot, j])

        for j in range(GRP):
            rb = j % NB
            gather_wait(rb, srcb.at[slot, j])

            @pl.loop(0, CHUNK, unroll=4)
            def _scale(e):
                sv = jnp.full((16,), slot, jnp.int32)
                jv = jnp.full((16,), j, jnp.int32)
                wv = plsc.load_gather(
                    wb, [sv, jv, jnp.full((16,), e, jnp.int32)])
                for k in range(D // 16):
                    sl = pl.ds(k * 16, 16)
                    rowsb[rb, e, sl] = rowsb[rb, e, sl] * wv

            scat_start(rb, dstb.at[slot, j])
            if j >= 1:
                pb = (j - 1) % NB
                scat_wait(pb, dstb.at[slot, j - 1])
                if j - 1 + NB < GRP:
                    gather_start(pb, srcb.at[slot, j - 1 + NB])

        scat_wait((GRP - 1) % NB, dstb.at[slot, GRP - 1])

    @pl.loop(0, NGRP // 2)
    def _pair(t):
        for parity in range(2):
            group(t * 2 + parity, parity)

    plsc.subcore_barrier()
    pltpu.sync_copy(acc.at[pl.ds(r0, ROWS_PER_TILE)],
                    out.at[pl.ds(c * HALF_PAD + r0, ROWS_PER_TILE)])


@functools.partial(
    pl.kernel,
    out_type=jax.ShapeDtypeStruct((BATCH,), jnp.float32),
    mesh=_mesh,
    compiler_params=pltpu.CompilerParams(needs_layout_passes=False,
                                         use_tc_tiling_on_sc=False),
    scratch_types=[
        pltpu.VMEM((BATCH // (NSC * NTILE),), jnp.int32),
        pltpu.VMEM((BATCH // (NSC * NTILE),), jnp.int32),
        pltpu.VMEM((BATCH // (NSC * NTILE), D), jnp.float32),
        pltpu.VMEM((BATCH // (NSC * NTILE), D), jnp.float32),
        pltpu.VMEM((BATCH // (NSC * NTILE), D), jnp.float32),
        pltpu.VMEM((BATCH // (NSC * NTILE), D), jnp.float32),
        pltpu.VMEM((BATCH // (NSC * NTILE), D), jnp.float32),
        pltpu.VMEM((BATCH // (NSC * NTILE), D), jnp.float32),
        pltpu.VMEM((BATCH // (NSC * NTILE),), jnp.float32),
        pltpu.SemaphoreType.DMA,
    ],
)
def _final_step(e0, e1, e2, uix_hbm, iix_hbm, out, uix, iix, u0, u1, u2,
                i0, i1, i2, outb, sem):
    c = lax.axis_index("c")
    s = lax.axis_index("s")
    wid = s * NSC + c
    per = BATCH // (NSC * NTILE)
    qbase = wid * per

    pltpu.sync_copy(uix_hbm.at[pl.ds(qbase, per)], uix)
    pltpu.sync_copy(iix_hbm.at[pl.ds(qbase, per)], iix)

    cps = [
        pltpu.async_copy(e0.at[uix], u0, sem),
        pltpu.async_copy(e1.at[uix], u1, sem),
        pltpu.async_copy(e2.at[uix], u2, sem),
        pltpu.async_copy(e0.at[iix], i0, sem),
        pltpu.async_copy(e1.at[iix], i1, sem),
        pltpu.async_copy(e2.at[iix], i2, sem),
    ]
    for cp in cps:
        cp.wait()

    lanes = lax.iota(jnp.int32, 16)

    @pl.loop(0, per // 16)
    def _group(g):
        gvec = jnp.zeros((16,), jnp.float32)
        for q in range(16):
            e = g * 16 + q
            dv = jnp.zeros((16,), jnp.float32)
            for j in range(D // 16):
                sl = pl.ds(j * 16, 16)
                su = (ALPHA[0] * u0[e, sl] + ALPHA[1] * u1[e, sl]
                      + ALPHA[2] * u2[e, sl])
                si = (ALPHA[0] * i0[e, sl] + ALPHA[1] * i1[e, sl]
                      + ALPHA[2] * i2[e, sl])
                dv = dv + su * si
            gamma = jnp.sum(dv)
            gvec = jnp.where(lanes == q, gamma, gvec)
        sig = 1.0 / (1.0 + jnp.exp(-gvec))
        outb[pl.ds(g * 16, 16)] = sig

    pltpu.sync_copy(outb, out.at[pl.ds(qbase, per)])


@jax.jit
def kernel(user, item, edge_index, edge_weight, user_emb, item_emb):
    zpad = jnp.zeros((HALF_PAD - HALF, D), jnp.float32)
    e0 = jnp.concatenate([user_emb, zpad, item_emb, zpad], axis=0)

    dst = edge_index[0].astype(jnp.int32)
    src = edge_index[1].astype(jnp.int32)
    w = edge_weight.astype(jnp.float32)
    pad = EDGES_PAD - N_EDGES
    dst = jnp.concatenate([dst, jnp.full((pad,), N_NODES + 8192, jnp.int32)])
    src = jnp.concatenate([src, jnp.zeros((pad,), jnp.int32)])
    w = jnp.concatenate([w, jnp.zeros((pad,), jnp.float32)])
    dstr = dst.reshape(NTILE, NGRP, GRP, CHUNK)
    srcr = src.reshape(NTILE, NGRP, GRP, CHUNK)
    wr = w.reshape(NTILE, NGRP, GRP, CHUNK)
    zrows = jnp.zeros((ROWS_PER_TILE, D), jnp.float32)

    e1 = _layer_step(e0, dstr, srcr, wr, zrows)
    e2 = _layer_step(e1, dstr, srcr, wr, zrows)

    uix = user.astype(jnp.int32)
    iix = item.astype(jnp.int32) + HALF_PAD
    return _final_step(e0, e1, e2, uix, iix)


# speedup vs baseline: 5.1877x; 1.4336x over previous
import functools

from jax import lax as _lax

_GDN = _lax.GatherDimensionNumbers(
    offset_dims=(), collapsed_slice_dims=(0,), start_index_map=(0,))


def _bcast_lane(vec, idx):
    return _lax.gather(vec, idx[:, None], _GDN, (1,),
                       mode=_lax.GatherScatterMode.PROMISE_IN_BOUNDS)

import jax
import jax.numpy as jnp
from jax import lax
from jax.experimental import pallas as pl
from jax.experimental.pallas import tpu as pltpu
from jax.experimental.pallas import tpu_sc as plsc

USER_NUM = 25000
ITEM_NUM = 25000
N_NODES = USER_NUM + ITEM_NUM
N_EDGES = 800000
D = 64
ALPHA = (0.3334, 0.3333, 0.3333)
BATCH = 4096

NSC = 2
NTILE = 16
HALF = 25000
HALF_PAD = 25088
NPAD = 2 * HALF_PAD
ROWS_PER_TILE = HALF_PAD // NTILE
CHUNK = 128
GRP = 7
NGRP = 56
EDGES_PER_TILE = NGRP * GRP * CHUNK
EDGES_PAD = NTILE * EDGES_PER_TILE
NB = 3

_mesh = plsc.VectorSubcoreMesh(core_axis_name="c", subcore_axis_name="s")


@functools.partial(
    pl.kernel,
    out_type=jax.ShapeDtypeStruct((NPAD, D), jnp.float32),
    mesh=_mesh,
    compiler_params=pltpu.CompilerParams(needs_layout_passes=False,
                                         use_tc_tiling_on_sc=False),
    scratch_types=[
        pltpu.VMEM((2, GRP, CHUNK), jnp.int32),
        pltpu.VMEM((2, GRP, CHUNK), jnp.int32),
        pltpu.VMEM((2, GRP, CHUNK), jnp.float32),
        pltpu.VMEM((NB, CHUNK, D), jnp.float32),
        pltpu.VMEM_SHARED((HALF_PAD, D), jnp.float32),
        pltpu.SemaphoreType.DMA,
        pltpu.SemaphoreType.DMA((NB,)),
        pltpu.SemaphoreType.DMA((NB,)),
    ],
)
def _layer_step(emb, dstr, srcr, wr, zrows, out, dstb, srcb, wb, rowsb, acc,
                isem, gsem, ssem):
    c = lax.axis_index("c")
    s = lax.axis_index("s")
    base = c * HALF
    r0 = s * ROWS_PER_TILE

    def idx_start(g, slot):
        pltpu.async_copy(dstr.at[s, g], dstb.at[slot], isem)
        pltpu.async_copy(srcr.at[s, g], srcb.at[slot], isem)
        pltpu.async_copy(wr.at[s, g], wb.at[slot], isem)

    def idx_wait(slot):
        pltpu.make_async_copy(dstr.at[s, 0], dstb.at[slot], isem).wait()
        pltpu.make_async_copy(srcr.at[s, 0], srcb.at[slot], isem).wait()
        pltpu.make_async_copy(wr.at[s, 0], wb.at[slot], isem).wait()

    def gather_start(slot, srow):
        pltpu.async_copy(emb.at[srow], rowsb.at[slot], gsem.at[slot])

    def gather_wait(slot, srow):
        pltpu.make_async_copy(emb.at[srow], rowsb.at[slot],
                              gsem.at[slot]).wait()

    def scat_start(slot, drow):
        pltpu.async_copy(rowsb.at[slot], acc.at[drow], ssem.at[slot],
                         add=True)

    def scat_wait(slot, drow):
        pltpu.make_async_copy(rowsb.at[slot], acc.at[drow],
                              ssem.at[slot]).wait()

    pltpu.sync_copy(zrows, acc.at[pl.ds(r0, ROWS_PER_TILE)])
    plsc.subcore_barrier()

    idx_start(0, 0)

    def group(g, slot):
        idx_wait(slot)
        nxt = pl.when(g + 1 < NGRP)(lambda: idx_start(g + 1, slot ^ 1))
        del nxt

        for j in range(GRP):
            for k in range(CHUNK // 16):
                sl = pl.ds(k * 16, 16)
                d = dstb[slot, j, sl] - base
                ok = (d >= 0) & (d < HALF)
                dstb[slot, j, sl] = jnp.where(ok, d, HALF)
                sv = srcb[slot, j, sl]
                srcb[slot, j, sl] = jnp.where(sv >= HALF,
                                              sv + (HALF_PAD - HALF), sv)
            if j < NB:
                gather_start(j, srcb.at[slot, j])

        for j in range(GRP):
            rb = j % NB
            gather_wait(rb, srcb.at[slot, j])

            @pl.loop(0, CHUNK // 16)
            def _scale(q):
                wvec = wb[slot, j, pl.ds(q * 16, 16)]
                lane = jnp.zeros((16,), jnp.int32)
                for r in range(16):
                    wv = _bcast_lane(wvec, lane)
                    e = q * 16 + r
                    vals = [rowsb[rb, e, pl.ds(k * 16, 16)]
                            for k in range(D // 16)]
                    prods = [v * wv for v in vals]
                    for k in range(D // 16):
                        rowsb[rb, e, pl.ds(k * 16, 16)] = prods[k]
                    if r < 15:
                        lane = lane + 1

            scat_start(rb, dstb.at[slot, j])
            if j >= 1:
                pb = (j - 1) % NB
                scat_wait(pb, dstb.at[slot, j - 1])
                if j - 1 + NB < GRP:
                    gather_start(pb, srcb.at[slot, j - 1 + NB])

        scat_wait((GRP - 1) % NB, dstb.at[slot, GRP - 1])

    @pl.loop(0, NGRP // 2)
    def _pair(t):
        for parity in range(2):
            group(t * 2 + parity, parity)

    plsc.subcore_barrier()
    pltpu.sync_copy(acc.at[pl.ds(r0, ROWS_PER_TILE)],
                    out.at[pl.ds(c * HALF_PAD + r0, ROWS_PER_TILE)])


@functools.partial(
    pl.kernel,
    out_type=jax.ShapeDtypeStruct((BATCH,), jnp.float32),
    mesh=_mesh,
    compiler_params=pltpu.CompilerParams(needs_layout_passes=False,
                                         use_tc_tiling_on_sc=False),
    scratch_types=[
        pltpu.VMEM((BATCH // (NSC * NTILE),), jnp.int32),
        pltpu.VMEM((BATCH // (NSC * NTILE),), jnp.int32),
        pltpu.VMEM((BATCH // (NSC * NTILE), D), jnp.float32),
        pltpu.VMEM((BATCH // (NSC * NTILE), D), jnp.float32),
        pltpu.VMEM((BATCH // (NSC * NTILE), D), jnp.float32),
        pltpu.VMEM((BATCH // (NSC * NTILE), D), jnp.float32),
        pltpu.VMEM((BATCH // (NSC * NTILE), D), jnp.float32),
        pltpu.VMEM((BATCH // (NSC * NTILE), D), jnp.float32),
        pltpu.VMEM((BATCH // (NSC * NTILE),), jnp.float32),
        pltpu.SemaphoreType.DMA,
    ],
)
def _final_step(e0, e1, e2, uix_hbm, iix_hbm, out, uix, iix, u0, u1, u2,
                i0, i1, i2, outb, sem):
    c = lax.axis_index("c")
    s = lax.axis_index("s")
    wid = s * NSC + c
    per = BATCH // (NSC * NTILE)
    qbase = wid * per

    pltpu.sync_copy(uix_hbm.at[pl.ds(qbase, per)], uix)
    pltpu.sync_copy(iix_hbm.at[pl.ds(qbase, per)], iix)

    cps = [
        pltpu.async_copy(e0.at[uix], u0, sem),
        pltpu.async_copy(e1.at[uix], u1, sem),
        pltpu.async_copy(e2.at[uix], u2, sem),
        pltpu.async_copy(e0.at[iix], i0, sem),
        pltpu.async_copy(e1.at[iix], i1, sem),
        pltpu.async_copy(e2.at[iix], i2, sem),
    ]
    for cp in cps:
        cp.wait()

    lanes = lax.iota(jnp.int32, 16)

    @pl.loop(0, per // 16)
    def _group(g):
        gvec = jnp.zeros((16,), jnp.float32)
        for q in range(16):
            e = g * 16 + q
            dv = jnp.zeros((16,), jnp.float32)
            for j in range(D // 16):
                sl = pl.ds(j * 16, 16)
                su = (ALPHA[0] * u0[e, sl] + ALPHA[1] * u1[e, sl]
                      + ALPHA[2] * u2[e, sl])
                si = (ALPHA[0] * i0[e, sl] + ALPHA[1] * i1[e, sl]
                      + ALPHA[2] * i2[e, sl])
                dv = dv + su * si
            gamma = jnp.sum(dv)
            gvec = jnp.where(lanes == q, gamma, gvec)
        sig = 1.0 / (1.0 + jnp.exp(-gvec))
        outb[pl.ds(g * 16, 16)] = sig

    pltpu.sync_copy(outb, out.at[pl.ds(qbase, per)])


@jax.jit
def kernel(user, item, edge_index, edge_weight, user_emb, item_emb):
    zpad = jnp.zeros((HALF_PAD - HALF, D), jnp.float32)
    e0 = jnp.concatenate([user_emb, zpad, item_emb, zpad], axis=0)

    dst = edge_index[0].astype(jnp.int32)
    src = edge_index[1].astype(jnp.int32)
    w = edge_weight.astype(jnp.float32)
    pad = EDGES_PAD - N_EDGES
    dst = jnp.concatenate([dst, jnp.full((pad,), N_NODES + 8192, jnp.int32)])
    src = jnp.concatenate([src, jnp.zeros((pad,), jnp.int32)])
    w = jnp.concatenate([w, jnp.zeros((pad,), jnp.float32)])
    dstr = dst.reshape(NTILE, NGRP, GRP, CHUNK)
    srcr = src.reshape(NTILE, NGRP, GRP, CHUNK)
    wr = w.reshape(NTILE, NGRP, GRP, CHUNK)
    zrows = jnp.zeros((ROWS_PER_TILE, D), jnp.float32)

    e1 = _layer_step(e0, dstr, srcr, wr, zrows)
    e2 = _layer_step(e1, dstr, srcr, wr, zrows)

    uix = user.astype(jnp.int32)
    iix = item.astype(jnp.int32) + HALF_PAD
    return _final_step(e0, e1, e2, uix, iix)
